# baseline (speedup 1.0000x reference)
import jax
import jax.numpy as jnp
from jax.experimental import pallas as pl
from jax.experimental.pallas import tpu as pltpu


def _fused_mlp_kernel(x_ref, w1_ref, b1_ref, w2_ref, b2_ref, o_ref):
    xb = x_ref[...].astype(jnp.bfloat16)
    h = jnp.dot(xb, w1_ref[...], preferred_element_type=jnp.float32)
    h = jnp.maximum(h + b1_ref[...], 0.0)
    out = jnp.dot(h.astype(jnp.bfloat16), w2_ref[...],
                  preferred_element_type=jnp.float32) + b2_ref[...]
    o_ref[...] = out.astype(o_ref.dtype)


def kernel(x, w1, b1, w2, b2):
    I = x.shape[-1]
    H = w1.shape[1]
    O = w2.shape[1]
    lead_shape = x.shape[:-1]

    x2 = x.reshape(-1, I)
    M = x2.shape[0]

    if M <= 256:
        tm = M
    else:
        tm = 256
    grid_m = pl.cdiv(M, tm)

    w1b = w1.astype(jnp.bfloat16)
    w2b = w2.astype(jnp.bfloat16)
    b1r = b1.reshape(1, H)
    b2r = b2.reshape(1, O)

    working = (2 * (I * H + H * O)
               + 2 * 4 * (tm * I + tm * O)
               + 4 * (tm * H + H + O))
    vmem_limit = int(min(max(2 * working, 4 * 1024 * 1024), 56 * 1024 * 1024))

    cost = pl.CostEstimate(
        flops=2 * M * (I * H + H * O),
        transcendentals=0,
        bytes_accessed=(4 * M * I + 2 * I * H + 4 * H
                        + 2 * H * O + 4 * O + 4 * M * O),
    )

    out = pl.pallas_call(
        _fused_mlp_kernel,
        out_shape=jax.ShapeDtypeStruct((M, O), x.dtype),
        grid=(grid_m,),
        in_specs=[
            pl.BlockSpec((tm, I), lambda i: (i, 0)),
            pl.BlockSpec((I, H), lambda i: (0, 0)),
            pl.BlockSpec((1, H), lambda i: (0, 0)),
            pl.BlockSpec((H, O), lambda i: (0, 0)),
            pl.BlockSpec((1, O), lambda i: (0, 0)),
        ],
        out_specs=pl.BlockSpec((tm, O), lambda i: (i, 0)),
        compiler_params=pltpu.CompilerParams(
            dimension_semantics=("parallel",),
            vmem_limit_bytes=vmem_limit,
        ),
        cost_estimate=cost,
    )(x2, w1b, b1r, w2b, b2r)

    return out.reshape(*lead_shape, O)


# speedup vs baseline: 1.1133x; 1.1133x over previous
import jax
import jax.numpy as jnp
from jax.experimental import pallas as pl
from jax.experimental.pallas import tpu as pltpu

_N_WTILES = 4


def _fused_mlp_kernel(x_ref, w1_hbm, b1_ref, w2_hbm, b2_ref, o_ref,
                      w1s, w2s, stg1, stg2, sem1, sem2):
    I, H = w1s.shape
    O = w2s.shape[1]
    r1 = I // _N_WTILES
    r2 = H // _N_WTILES

    @pl.when(pl.program_id(0) == 0)
    def _load_and_cast_weights():
        def copy1(t):
            return pltpu.make_async_copy(
                w1_hbm.at[pl.ds(t * r1, r1), :], stg1.at[t % 2], sem1.at[t % 2])

        def copy2(t):
            return pltpu.make_async_copy(
                w2_hbm.at[pl.ds(t * r2, r2), :], stg2.at[t % 2], sem2.at[t % 2])

        copy1(0).start()
        copy1(1).start()
        for t in range(_N_WTILES):
            copy1(t).wait()
            if t + 2 < _N_WTILES:
                copy1(t + 2).start()
            elif t + 2 - _N_WTILES < 2:
                copy2(t + 2 - _N_WTILES).start()
            w1s[pl.ds(t * r1, r1), :] = stg1[t % 2].astype(jnp.bfloat16)
        for t in range(_N_WTILES):
            copy2(t).wait()
            if t + 2 < _N_WTILES:
                copy2(t + 2).start()
            w2s[pl.ds(t * r2, r2), :] = stg2[t % 2].astype(jnp.bfloat16)

    xb = x_ref[...].astype(jnp.bfloat16)
    h = jnp.dot(xb, w1s[...], preferred_element_type=jnp.float32)
    h = jnp.maximum(h + b1_ref[...], 0.0)
    out = jnp.dot(h.astype(jnp.bfloat16), w2s[...],
                  preferred_element_type=jnp.float32) + b2_ref[...]
    o_ref[...] = out.astype(o_ref.dtype)


def kernel(x, w1, b1, w2, b2):
    I = x.shape[-1]
    H = w1.shape[1]
    O = w2.shape[1]
    lead_shape = x.shape[:-1]

    x2 = x.reshape(-1, I)
    M = x2.shape[0]

    if M <= 256:
        tm = M
    else:
        tm = 256
    grid_m = pl.cdiv(M, tm)

    b1r = b1.reshape(1, H)
    b2r = b2.reshape(1, O)

    r1 = I // _N_WTILES
    r2 = H // _N_WTILES

    working = (2 * (I * H + H * O)
               + 4 * 2 * (r1 * H + r2 * O)
               + 2 * 4 * (tm * I + tm * O)
               + 4 * (tm * H + H + O))
    vmem_limit = int(min(max(working + 8 * 1024 * 1024, 4 * 1024 * 1024),
                         56 * 1024 * 1024))

    cost = pl.CostEstimate(
        flops=2 * M * (I * H + H * O),
        transcendentals=0,
        bytes_accessed=4 * (M * I + I * H + H + H * O + O + M * O),
    )

    out = pl.pallas_call(
        _fused_mlp_kernel,
        out_shape=jax.ShapeDtypeStruct((M, O), x.dtype),
        grid=(grid_m,),
        in_specs=[
            pl.BlockSpec((tm, I), lambda i: (i, 0)),
            pl.BlockSpec(memory_space=pl.ANY),
            pl.BlockSpec((1, H), lambda i: (0, 0)),
            pl.BlockSpec(memory_space=pl.ANY),
            pl.BlockSpec((1, O), lambda i: (0, 0)),
        ],
        out_specs=pl.BlockSpec((tm, O), lambda i: (i, 0)),
        scratch_shapes=[
            pltpu.VMEM((I, H), jnp.bfloat16),
            pltpu.VMEM((H, O), jnp.bfloat16),
            pltpu.VMEM((2, r1, H), jnp.float32),
            pltpu.VMEM((2, r2, O), jnp.float32),
            pltpu.SemaphoreType.DMA((2,)),
            pltpu.SemaphoreType.DMA((2,)),
        ],
        compiler_params=pltpu.CompilerParams(
            dimension_semantics=("arbitrary",),
            vmem_limit_bytes=vmem_limit,
        ),
        cost_estimate=cost,
    )(x2, w1, b1r, w2, b2r)

    return out.reshape(*lead_shape, O)


# speedup vs baseline: 1.1465x; 1.0298x over previous
import jax
import jax.numpy as jnp
from jax.experimental import pallas as pl
from jax.experimental.pallas import tpu as pltpu

_NT = 4


def _mlp_kernel(x_ref, w1_hbm, b1_ref, w2_hbm, b2_ref, o_ref,
                w1s, w2s, sem1, sem2):
    I, H = w1s.shape
    O = w2s.shape[1]
    r1 = I // _NT
    r2 = H // _NT
    i = pl.program_id(0)

    def c1(t):
        return pltpu.make_async_copy(
            w1_hbm.at[pl.ds(t * r1, r1), :], w1s.at[pl.ds(t * r1, r1), :],
            sem1.at[t])

    def c2(t):
        return pltpu.make_async_copy(
            w2_hbm.at[pl.ds(t * r2, r2), :], w2s.at[pl.ds(t * r2, r2), :],
            sem2.at[t])

    @pl.when(i == 0)
    def _first_step():
        for t in range(_NT):
            c1(t).start()
        for t in range(_NT):
            c2(t).start()
        x = x_ref[...]
        h = b1_ref[...] * jnp.ones((x.shape[0], 1), jnp.float32)
        for t in range(_NT):
            c1(t).wait()
            h = h + jnp.dot(x[:, t * r1:(t + 1) * r1],
                            w1s[pl.ds(t * r1, r1), :],
                            preferred_element_type=jnp.float32)
        h = jnp.maximum(h, 0.0)
        acc = b2_ref[...] * jnp.ones((x.shape[0], 1), jnp.float32)
        for t in range(_NT):
            c2(t).wait()
            acc = acc + jnp.dot(h[:, t * r2:(t + 1) * r2],
                                w2s[pl.ds(t * r2, r2), :],
                                preferred_element_type=jnp.float32)
        o_ref[...] = acc.astype(o_ref.dtype)

    @pl.when(i > 0)
    def _steady_state():
        h = jnp.dot(x_ref[...], w1s[...], preferred_element_type=jnp.float32)
        h = jnp.maximum(h + b1_ref[...], 0.0)
        out = jnp.dot(h, w2s[...],
                      preferred_element_type=jnp.float32) + b2_ref[...]
        o_ref[...] = out.astype(o_ref.dtype)


def kernel(x, w1, b1, w2, b2):
    I = x.shape[-1]
    H = w1.shape[1]
    O = w2.shape[1]
    lead_shape = x.shape[:-1]

    x2 = x.reshape(-1, I)
    M = x2.shape[0]

    if M <= 256:
        tm = M
    else:
        tm = 256
    grid_m = pl.cdiv(M, tm)

    b1r = b1.reshape(1, H)
    b2r = b2.reshape(1, O)

    working = (4 * (I * H + H * O)
               + 2 * 4 * (tm * I + tm * O)
               + 4 * (tm * H + H + O))
    vmem_limit = int(min(max(working + 8 * 1024 * 1024, 4 * 1024 * 1024),
                         56 * 1024 * 1024))

    cost = pl.CostEstimate(
        flops=2 * M * (I * H + H * O),
        transcendentals=0,
        bytes_accessed=4 * (M * I + I * H + H + H * O + O + M * O),
    )

    out = pl.pallas_call(
        _mlp_kernel,
        out_shape=jax.ShapeDtypeStruct((M, O), x.dtype),
        grid=(grid_m,),
        in_specs=[
            pl.BlockSpec((tm, I), lambda i: (i, 0)),
            pl.BlockSpec(memory_space=pl.ANY),
            pl.BlockSpec((1, H), lambda i: (0, 0)),
            pl.BlockSpec(memory_space=pl.ANY),
            pl.BlockSpec((1, O), lambda i: (0, 0)),
        ],
        out_specs=pl.BlockSpec((tm, O), lambda i: (i, 0)),
        scratch_shapes=[
            pltpu.VMEM((I, H), jnp.float32),
            pltpu.VMEM((H, O), jnp.float32),
            pltpu.SemaphoreType.DMA((_NT,)),
            pltpu.SemaphoreType.DMA((_NT,)),
        ],
        compiler_params=pltpu.CompilerParams(
            dimension_semantics=("arbitrary",),
            vmem_limit_bytes=vmem_limit,
        ),
        cost_estimate=cost,
    )(x2, w1, b1r, w2, b2r)

    return out.reshape(*lead_shape, O)


# speedup vs baseline: 1.2177x; 1.0621x over previous
import jax
import jax.numpy as jnp
from jax.experimental import pallas as pl
from jax.experimental.pallas import tpu as pltpu

_NT = 4


def _mlp_kernel(x_ref, w1_hbm, b1_ref, w2_hbm, b2_ref, o_ref,
                w1s, w2s, sem1, sem2):
    I, H = w1s.shape
    O = w2s.shape[1]
    r1 = I // _NT
    r2 = H // _NT
    i = pl.program_id(0)

    def c1(t):
        return pltpu.make_async_copy(
            w1_hbm.at[pl.ds(t * r1, r1), :], w1s.at[pl.ds(t * r1, r1), :],
            sem1.at[t])

    def c2(t):
        return pltpu.make_async_copy(
            w2_hbm.at[pl.ds(t * r2, r2), :], w2s.at[pl.ds(t * r2, r2), :],
            sem2.at[t])

    @pl.when(i == 0)
    def _first_step():
        for t in range(_NT):
            c1(t).start()
        for t in range(_NT):
            c2(t).start()
        x = x_ref[...]
        h = b1_ref[...] * jnp.ones((x.shape[0], 1), jnp.float32)
        for t in range(_NT):
            c1(t).wait()
            h = h + jnp.dot(x[:, t * r1:(t + 1) * r1],
                            w1s[pl.ds(t * r1, r1), :],
                            preferred_element_type=jnp.float32)
        h = jnp.maximum(h, 0.0)
        acc = b2_ref[...] * jnp.ones((x.shape[0], 1), jnp.float32)
        for t in range(_NT):
            c2(t).wait()
            acc = acc + jnp.dot(h[:, t * r2:(t + 1) * r2],
                                w2s[pl.ds(t * r2, r2), :],
                                preferred_element_type=jnp.float32)
        o_ref[...] = acc.astype(o_ref.dtype)

    @pl.when(i > 0)
    def _steady_state():
        h = jnp.dot(x_ref[...], w1s[...], preferred_element_type=jnp.float32)
        h = jnp.maximum(h + b1_ref[...], 0.0)
        out = jnp.dot(h, w2s[...],
                      preferred_element_type=jnp.float32) + b2_ref[...]
        o_ref[...] = out.astype(o_ref.dtype)


def kernel(x, w1, b1, w2, b2):
    I = x.shape[-1]
    H = w1.shape[1]
    O = w2.shape[1]
    lead_shape = x.shape[:-1]

    x2 = x.reshape(-1, I)
    M = x2.shape[0]

    if M <= 512:
        tm = M
    else:
        tm = 512
    grid_m = pl.cdiv(M, tm)

    b1r = b1.reshape(1, H)
    b2r = b2.reshape(1, O)

    working = (4 * (I * H + H * O)
               + 2 * 4 * (tm * I + tm * O)
               + 4 * (tm * H + H + O))
    vmem_limit = int(min(max(working + 8 * 1024 * 1024, 4 * 1024 * 1024),
                         56 * 1024 * 1024))

    cost = pl.CostEstimate(
        flops=2 * M * (I * H + H * O),
        transcendentals=0,
        bytes_accessed=4 * (M * I + I * H + H + H * O + O + M * O),
    )

    out = pl.pallas_call(
        _mlp_kernel,
        out_shape=jax.ShapeDtypeStruct((M, O), x.dtype),
        grid=(grid_m,),
        in_specs=[
            pl.BlockSpec((tm, I), lambda i: (i, 0)),
            pl.BlockSpec(memory_space=pl.ANY),
            pl.BlockSpec((1, H), lambda i: (0, 0)),
            pl.BlockSpec(memory_space=pl.ANY),
            pl.BlockSpec((1, O), lambda i: (0, 0)),
        ],
        out_specs=pl.BlockSpec((tm, O), lambda i: (i, 0)),
        scratch_shapes=[
            pltpu.VMEM((I, H), jnp.float32),
            pltpu.VMEM((H, O), jnp.float32),
            pltpu.SemaphoreType.DMA((_NT,)),
            pltpu.SemaphoreType.DMA((_NT,)),
        ],
        compiler_params=pltpu.CompilerParams(
            dimension_semantics=("arbitrary",),
            vmem_limit_bytes=vmem_limit,
        ),
        cost_estimate=cost,
    )(x2, w1, b1r, w2, b2r)

    return out.reshape(*lead_shape, O)
